# baseline (speedup 1.0000x reference)
import functools

import jax
import jax.numpy as jnp
from jax import lax
from jax.experimental import pallas as pl
from jax.experimental.pallas import tpu as pltpu
from jax.experimental.pallas import tpu_sc as plsc

V, E, H = 10000, 512, 512
SRC_LEN, TRG_LEN, B = 50, 50, 64
G3 = 3 * H

_NC, _NS = 2, 16
_NW = _NC * _NS
_NIDX = SRC_LEN * B
_NPAD = 3328
_BPW = _NPAD // _NW


@functools.cache
def _make_sc_gather():
    @functools.partial(
        pl.kernel,
        mesh=plsc.VectorSubcoreMesh(core_axis_name="c", subcore_axis_name="s",
                                    num_cores=_NC),
        out_type=jax.ShapeDtypeStruct((_NPAD, E), jnp.float32),
        scratch_types=[
            pltpu.VMEM((_BPW,), jnp.int32),
            pltpu.VMEM((_BPW, E), jnp.float32),
            pltpu.SemaphoreType.DMA,
        ],
    )
    def _sc_gather(table_hbm, idx_hbm, out_hbm, idx_v, rows_v, sem):
        wid = lax.axis_index("s") * _NC + lax.axis_index("c")
        base = wid * _BPW
        pltpu.sync_copy(idx_hbm.at[pl.ds(base, _BPW)], idx_v)
        pltpu.async_copy(table_hbm.at[idx_v], rows_v, sem).wait()
        pltpu.sync_copy(rows_v, out_hbm.at[pl.ds(base, _BPW)])

    return _sc_gather


def _dot(a, b):
    return lax.dot_general(a.astype(jnp.bfloat16), b.astype(jnp.bfloat16),
                           (((1,), (0,)), ((), ())),
                           preferred_element_type=jnp.float32,
                           precision=lax.Precision.DEFAULT)


def _enc_body(xemb_ref, wx_ref, wh_ref, bx_ref, bh_ref, wa2_ref,
              eo_ref, ek_ref, hn_ref, h_s, gx_s):
    h_s[...] = jnp.zeros((B, H), jnp.float32)
    gx_s[...] = _dot(xemb_ref[...].reshape(SRC_LEN * B, E), wx_ref[...]) \
        + bx_ref[...]
    wh = wh_ref[...]
    bh = bh_ref[...]

    def step(t, carry):
        h = h_s[...]
        gx = gx_s[pl.ds(t * B, B), :]
        gh = _dot(h, wh) + bh
        r = jax.nn.sigmoid(gx[:, :H] + gh[:, :H])
        z = jax.nn.sigmoid(gx[:, H:2 * H] + gh[:, H:2 * H])
        n = jnp.tanh(gx[:, 2 * H:] + r * gh[:, 2 * H:])
        h2 = (1.0 - z) * n + z * h
        eo_ref[t] = h2
        ek_ref[t] = _dot(h2, wa2_ref[...])
        h_s[...] = h2
        return carry

    lax.fori_loop(0, SRC_LEN, step, 0)
    hn_ref[...] = h_s[...]


def _encoder(src_emb, enc_Wx, enc_Wh, enc_bx, enc_bh, attn_W2):
    full = lambda s: pl.BlockSpec(s, lambda: tuple(0 for _ in s))
    return pl.pallas_call(
        _enc_body,
        grid=(),
        in_specs=[
            full((SRC_LEN, B, E)),
            full((E, G3)),
            full((H, G3)),
            full((1, G3)),
            full((1, G3)),
            full((H, H)),
        ],
        out_specs=[
            full((SRC_LEN, B, H)),
            full((SRC_LEN, B, H)),
            full((B, H)),
        ],
        out_shape=[
            jax.ShapeDtypeStruct((SRC_LEN, B, H), jnp.float32),
            jax.ShapeDtypeStruct((SRC_LEN, B, H), jnp.float32),
            jax.ShapeDtypeStruct((B, H), jnp.float32),
        ],
        scratch_shapes=[
            pltpu.VMEM((B, H), jnp.float32),
            pltpu.VMEM((SRC_LEN * B, G3), jnp.float32),
        ],
    )(src_emb, enc_Wx, enc_Wh, enc_bx, enc_bh, attn_W2)


VT = 1024
NV = (V + VT - 1) // VT
VPAD = NV * VT
_ECH = 2000
_NB = 6
_AHEAD = 5


def _dec_body(tf_ref, trg_ref, hid_ref, etab_ref, eo_ref, ek_ref,
              w1_ref, av_ref, wxa_ref, wxb_ref, wh_ref, bx_ref, bh_ref,
              fcb_ref, fcw_hbm, out_ref,
              h_s, feat_s, tok_s, bmax_s, barg_s, fbuf, sem):
    t = pl.program_id(0)

    def start(tile, slot):
        pltpu.make_async_copy(fcw_hbm.at[tile], fbuf.at[slot],
                              sem.at[slot]).start()

    def wait(slot):
        pltpu.make_async_copy(fcw_hbm.at[0], fbuf.at[slot],
                              sem.at[slot]).wait()

    @pl.when(t == 0)
    def _init():
        out_ref[...] = jnp.zeros_like(out_ref)
        h_s[...] = hid_ref[...]
        tok_s[...] = trg_ref[0]

    @pl.when(t > 0)
    def _step():
        for k in range(_AHEAD - 1):
            start(k, k)

        tok = tok_s[...]
        emb = jnp.zeros((B, E), jnp.float32)
        for c in range(V // _ECH):
            io = lax.broadcasted_iota(jnp.int32, (B, _ECH), 1) + c * _ECH
            oh = (io == tok).astype(jnp.float32)
            emb = emb + _dot(oh, etab_ref[c * _ECH:(c + 1) * _ECH, :])
        h = h_s[...]
        q = _dot(h, w1_ref[...])
        en = jnp.tanh(ek_ref[...] + q[None])
        sc = jnp.sum(en * av_ref[...][None], axis=2)
        m = jnp.max(sc, axis=0, keepdims=True)
        e = jnp.exp(sc - m)
        a = e / jnp.sum(e, axis=0, keepdims=True)
        ctx = jnp.sum(a[:, :, None] * eo_ref[...], axis=0)
        gx = _dot(emb, wxa_ref[...]) + _dot(ctx, wxb_ref[...]) + bx_ref[...]
        gh = _dot(h, wh_ref[...]) + bh_ref[...]
        r = jax.nn.sigmoid(gx[:, :H] + gh[:, :H])
        z = jax.nn.sigmoid(gx[:, H:2 * H] + gh[:, H:2 * H])
        n = jnp.tanh(gx[:, 2 * H:] + r * gh[:, 2 * H:])
        h2 = (1.0 - z) * n + z * h
        h_s[...] = h2
        feat_s[:, :H] = h2
        feat_s[:, H:2 * H] = ctx
        feat_s[:, 2 * H:] = emb
        feat = feat_s[...]

        bmax = jnp.full((B, 1), -jnp.inf, jnp.float32)
        barg = jnp.zeros((B, 1), jnp.int32)
        for v in range(NV):
            if v + _AHEAD - 1 < NV:
                start(v + _AHEAD - 1, (v + _AHEAD - 1) % _NB)
            wait(v % _NB)
            w = VT if v < NV - 1 else V - (NV - 1) * VT
            logits = _dot(feat, fbuf[v % _NB]) \
                + fcb_ref[:, pl.ds(v * VT, VT)]
            logits = logits[:, :w]
            out_ref[0, :, pl.ds(v * VT, w)] = logits
            col = lax.broadcasted_iota(jnp.int32, (B, w), 1) + v * VT
            tmax = jnp.max(logits, axis=1, keepdims=True)
            targ = jnp.min(jnp.where(logits == tmax, col, jnp.int32(2 ** 30)),
                           axis=1, keepdims=True)
            better = tmax > bmax
            barg = jnp.where(better, targ, barg)
            bmax = jnp.maximum(tmax, bmax)
        bmax_s[...] = bmax
        barg_s[...] = barg

        tf = tf_ref[t]
        tok_s[...] = jnp.where(tf != 0, trg_ref[0], barg_s[...])


def _decoder(tf_i32, trg3, hidden, embedding, enc_outs, ek,
             attn_W1, attn_v2, wxa, wxb, dec_Wh, dec_bx2, dec_bh2,
             fcw3, fc_b2):
    cst = lambda s: pl.BlockSpec(s, lambda t, tf: tuple(0 for _ in s))
    grid_spec = pltpu.PrefetchScalarGridSpec(
        num_scalar_prefetch=1,
        grid=(TRG_LEN,),
        in_specs=[
            pl.BlockSpec((1, B, 1), lambda t, tf: (t, 0, 0)),
            cst((B, H)),
            cst((V, E)),
            cst((SRC_LEN, B, H)),
            cst((SRC_LEN, B, H)),
            cst((H, H)),
            cst((1, H)),
            cst((E, G3)),
            cst((H, G3)),
            cst((H, G3)),
            cst((1, G3)),
            cst((1, G3)),
            cst((1, VPAD)),
            pl.BlockSpec(memory_space=pl.ANY),
        ],
        out_specs=pl.BlockSpec((1, B, V), lambda t, tf: (t, 0, 0)),
        scratch_shapes=[
            pltpu.VMEM((B, H), jnp.float32),
            pltpu.VMEM((B, E + 2 * H), jnp.float32),
            pltpu.VMEM((B, 1), jnp.int32),
            pltpu.VMEM((B, 1), jnp.float32),
            pltpu.VMEM((B, 1), jnp.int32),
            pltpu.VMEM((_NB, E + 2 * H, VT), fcw3.dtype),
            pltpu.SemaphoreType.DMA((_NB,)),
        ],
    )
    return pl.pallas_call(
        _dec_body,
        grid_spec=grid_spec,
        out_shape=jax.ShapeDtypeStruct((TRG_LEN, B, V), jnp.float32),
        compiler_params=pltpu.CompilerParams(
            dimension_semantics=("arbitrary",)),
    )(tf_i32, trg3, hidden, embedding, enc_outs, ek,
      attn_W1, attn_v2, wxa, wxb, dec_Wh, dec_bx2, dec_bh2, fc_b2, fcw3)


def kernel(src, trg, tf_flags, embedding, enc_Wx, enc_Wh, enc_bx, enc_bh,
           attn_W, attn_v, dec_Wx, dec_Wh, dec_bx, dec_bh, fc_W, fc_b):
    src_flat = src.reshape(-1).astype(jnp.int32)
    src_pad = jnp.pad(src_flat, (0, _NPAD - _NIDX))
    gathered = _make_sc_gather()(embedding, src_pad)
    src_emb = gathered[:_NIDX].reshape(SRC_LEN, B, E)

    bf = jnp.bfloat16
    enc_outs, ek, hidden = _encoder(
        src_emb, enc_Wx.astype(bf), enc_Wh.astype(bf),
        enc_bx.reshape(1, G3), enc_bh.reshape(1, G3), attn_W[H:].astype(bf))

    out = _decoder(
        tf_flags.astype(jnp.int32),
        trg.astype(jnp.int32).reshape(TRG_LEN, B, 1),
        hidden, embedding.astype(bf), enc_outs, ek,
        attn_W[:H].astype(bf), attn_v.reshape(1, H),
        dec_Wx[:E].astype(bf), dec_Wx[E:].astype(bf), dec_Wh.astype(bf),
        dec_bx.reshape(1, G3), dec_bh.reshape(1, G3),
        jnp.pad(fc_W.astype(bf), ((0, 0), (0, VPAD - V)))
           .reshape(E + 2 * H, NV, VT).transpose(1, 0, 2),
        jnp.pad(fc_b, (0, VPAD - V)).reshape(1, VPAD))
    return out


# speedup vs baseline: 1.1149x; 1.1149x over previous
import functools

import jax
import jax.numpy as jnp
from jax import lax
from jax.experimental import pallas as pl
from jax.experimental.pallas import tpu as pltpu
from jax.experimental.pallas import tpu_sc as plsc

V, E, H = 10000, 512, 512
SRC_LEN, TRG_LEN, B = 50, 50, 64
G3 = 3 * H

_NC, _NS = 2, 16
_NW = _NC * _NS
_NIDX = SRC_LEN * B
_NPAD = 3328
_BPW = _NPAD // _NW


@functools.cache
def _make_sc_gather():
    @functools.partial(
        pl.kernel,
        mesh=plsc.VectorSubcoreMesh(core_axis_name="c", subcore_axis_name="s",
                                    num_cores=_NC),
        out_type=jax.ShapeDtypeStruct((_NPAD, E), jnp.float32),
        scratch_types=[
            pltpu.VMEM((_BPW,), jnp.int32),
            pltpu.VMEM((_BPW, E), jnp.float32),
            pltpu.SemaphoreType.DMA,
        ],
    )
    def _sc_gather(table_hbm, idx_hbm, out_hbm, idx_v, rows_v, sem):
        wid = lax.axis_index("s") * _NC + lax.axis_index("c")
        base = wid * _BPW
        pltpu.sync_copy(idx_hbm.at[pl.ds(base, _BPW)], idx_v)
        pltpu.async_copy(table_hbm.at[idx_v], rows_v, sem).wait()
        pltpu.sync_copy(rows_v, out_hbm.at[pl.ds(base, _BPW)])

    return _sc_gather


def _dot(a, b):
    return lax.dot_general(a.astype(jnp.bfloat16), b.astype(jnp.bfloat16),
                           (((1,), (0,)), ((), ())),
                           preferred_element_type=jnp.float32,
                           precision=lax.Precision.DEFAULT)


def _enc_body(xemb_ref, wx_ref, wh_ref, bx_ref, bh_ref, wa2_ref,
              eo_ref, ek_ref, hn_ref, h_s, gx_s):
    h_s[...] = jnp.zeros((B, H), jnp.float32)
    gx_s[...] = _dot(xemb_ref[...].reshape(SRC_LEN * B, E), wx_ref[...]) \
        + bx_ref[...]
    wh = wh_ref[...]
    bh = bh_ref[...]

    def step(t, carry):
        h = h_s[...]
        gx = gx_s[pl.ds(t * B, B), :]
        gh = _dot(h, wh) + bh
        r = jax.nn.sigmoid(gx[:, :H] + gh[:, :H])
        z = jax.nn.sigmoid(gx[:, H:2 * H] + gh[:, H:2 * H])
        n = jnp.tanh(gx[:, 2 * H:] + r * gh[:, 2 * H:])
        h2 = (1.0 - z) * n + z * h
        eo_ref[t] = h2
        ek_ref[t] = _dot(h2, wa2_ref[...])
        h_s[...] = h2
        return carry

    lax.fori_loop(0, SRC_LEN, step, 0)
    hn_ref[...] = h_s[...]


def _encoder(src_emb, enc_Wx, enc_Wh, enc_bx, enc_bh, attn_W2):
    full = lambda s: pl.BlockSpec(s, lambda: tuple(0 for _ in s))
    return pl.pallas_call(
        _enc_body,
        grid=(),
        in_specs=[
            full((SRC_LEN, B, E)),
            full((E, G3)),
            full((H, G3)),
            full((1, G3)),
            full((1, G3)),
            full((H, H)),
        ],
        out_specs=[
            full((SRC_LEN, B, H)),
            full((SRC_LEN, B, H)),
            full((B, H)),
        ],
        out_shape=[
            jax.ShapeDtypeStruct((SRC_LEN, B, H), jnp.float32),
            jax.ShapeDtypeStruct((SRC_LEN, B, H), jnp.float32),
            jax.ShapeDtypeStruct((B, H), jnp.float32),
        ],
        scratch_shapes=[
            pltpu.VMEM((B, H), jnp.float32),
            pltpu.VMEM((SRC_LEN * B, G3), jnp.float32),
        ],
    )(src_emb, enc_Wx, enc_Wh, enc_bx, enc_bh, attn_W2)


VT = 2048
NV = (V + VT - 1) // VT
VPAD = NV * VT
_ECH = 2000
_NB = 4
_AHEAD = 3


def _dec_body(tf_ref, trg_ref, hid_ref, etab_ref, eo_ref, ek_ref,
              w1_ref, av_ref, wxa_ref, wxb_ref, wh_ref, bx_ref, bh_ref,
              fcb_ref, fcw_hbm, out_ref,
              h_s, feat_s, tok_s, bmax_s, barg_s, fbuf, sem):
    t = pl.program_id(0)

    def start(tile, slot):
        pltpu.make_async_copy(fcw_hbm.at[tile], fbuf.at[slot],
                              sem.at[slot]).start()

    def wait(slot):
        pltpu.make_async_copy(fcw_hbm.at[0], fbuf.at[slot],
                              sem.at[slot]).wait()

    @pl.when(t == 0)
    def _init():
        out_ref[...] = jnp.zeros_like(out_ref)
        h_s[...] = hid_ref[...]
        tok_s[...] = trg_ref[0]

    @pl.when(t > 0)
    def _step():
        for k in range(_AHEAD - 1):
            start(k, k)

        tok = tok_s[...]
        emb = jnp.zeros((B, E), jnp.float32)
        for c in range(V // _ECH):
            io = lax.broadcasted_iota(jnp.int32, (B, _ECH), 1) + c * _ECH
            oh = (io == tok).astype(jnp.float32)
            emb = emb + _dot(oh, etab_ref[c * _ECH:(c + 1) * _ECH, :])
        h = h_s[...]
        q = _dot(h, w1_ref[...])
        en = jnp.tanh(ek_ref[...] + q[None])
        sc = jnp.sum(en * av_ref[...][None], axis=2)
        m = jnp.max(sc, axis=0, keepdims=True)
        e = jnp.exp(sc - m)
        a = e / jnp.sum(e, axis=0, keepdims=True)
        ctx = jnp.sum(a[:, :, None] * eo_ref[...], axis=0)
        gx = _dot(emb, wxa_ref[...]) + _dot(ctx, wxb_ref[...]) + bx_ref[...]
        gh = _dot(h, wh_ref[...]) + bh_ref[...]
        r = jax.nn.sigmoid(gx[:, :H] + gh[:, :H])
        z = jax.nn.sigmoid(gx[:, H:2 * H] + gh[:, H:2 * H])
        n = jnp.tanh(gx[:, 2 * H:] + r * gh[:, 2 * H:])
        h2 = (1.0 - z) * n + z * h
        h_s[...] = h2
        feat_s[:, :H] = h2
        feat_s[:, H:2 * H] = ctx
        feat_s[:, 2 * H:] = emb
        feat = feat_s[...]

        bmax = jnp.full((B, 1), -jnp.inf, jnp.float32)
        barg = jnp.zeros((B, 1), jnp.int32)
        for v in range(NV):
            if v + _AHEAD - 1 < NV:
                start(v + _AHEAD - 1, (v + _AHEAD - 1) % _NB)
            wait(v % _NB)
            w = VT if v < NV - 1 else V - (NV - 1) * VT
            logits = _dot(feat, fbuf[v % _NB]) \
                + fcb_ref[:, pl.ds(v * VT, VT)]
            logits = logits[:, :w]
            out_ref[0, :, pl.ds(v * VT, w)] = logits
            col = lax.broadcasted_iota(jnp.int32, (B, w), 1) + v * VT
            tmax = jnp.max(logits, axis=1, keepdims=True)
            targ = jnp.min(jnp.where(logits == tmax, col, jnp.int32(2 ** 30)),
                           axis=1, keepdims=True)
            better = tmax > bmax
            barg = jnp.where(better, targ, barg)
            bmax = jnp.maximum(tmax, bmax)
        bmax_s[...] = bmax
        barg_s[...] = barg

        tf = tf_ref[t]
        tok_s[...] = jnp.where(tf != 0, trg_ref[0], barg_s[...])


def _decoder(tf_i32, trg3, hidden, embedding, enc_outs, ek,
             attn_W1, attn_v2, wxa, wxb, dec_Wh, dec_bx2, dec_bh2,
             fcw3, fc_b2):
    cst = lambda s: pl.BlockSpec(s, lambda t, tf: tuple(0 for _ in s))
    grid_spec = pltpu.PrefetchScalarGridSpec(
        num_scalar_prefetch=1,
        grid=(TRG_LEN,),
        in_specs=[
            pl.BlockSpec((1, B, 1), lambda t, tf: (t, 0, 0)),
            cst((B, H)),
            cst((V, E)),
            cst((SRC_LEN, B, H)),
            cst((SRC_LEN, B, H)),
            cst((H, H)),
            cst((1, H)),
            cst((E, G3)),
            cst((H, G3)),
            cst((H, G3)),
            cst((1, G3)),
            cst((1, G3)),
            cst((1, VPAD)),
            pl.BlockSpec(memory_space=pl.ANY),
        ],
        out_specs=pl.BlockSpec((1, B, V), lambda t, tf: (t, 0, 0)),
        scratch_shapes=[
            pltpu.VMEM((B, H), jnp.float32),
            pltpu.VMEM((B, E + 2 * H), jnp.float32),
            pltpu.VMEM((B, 1), jnp.int32),
            pltpu.VMEM((B, 1), jnp.float32),
            pltpu.VMEM((B, 1), jnp.int32),
            pltpu.VMEM((_NB, E + 2 * H, VT), fcw3.dtype),
            pltpu.SemaphoreType.DMA((_NB,)),
        ],
    )
    return pl.pallas_call(
        _dec_body,
        grid_spec=grid_spec,
        out_shape=jax.ShapeDtypeStruct((TRG_LEN, B, V), jnp.float32),
        compiler_params=pltpu.CompilerParams(
            dimension_semantics=("arbitrary",)),
    )(tf_i32, trg3, hidden, embedding, enc_outs, ek,
      attn_W1, attn_v2, wxa, wxb, dec_Wh, dec_bx2, dec_bh2, fc_b2, fcw3)


def kernel(src, trg, tf_flags, embedding, enc_Wx, enc_Wh, enc_bx, enc_bh,
           attn_W, attn_v, dec_Wx, dec_Wh, dec_bx, dec_bh, fc_W, fc_b):
    src_flat = src.reshape(-1).astype(jnp.int32)
    src_pad = jnp.pad(src_flat, (0, _NPAD - _NIDX))
    gathered = _make_sc_gather()(embedding, src_pad)
    src_emb = gathered[:_NIDX].reshape(SRC_LEN, B, E)

    bf = jnp.bfloat16
    enc_outs, ek, hidden = _encoder(
        src_emb, enc_Wx.astype(bf), enc_Wh.astype(bf),
        enc_bx.reshape(1, G3), enc_bh.reshape(1, G3), attn_W[H:].astype(bf))

    out = _decoder(
        tf_flags.astype(jnp.int32),
        trg.astype(jnp.int32).reshape(TRG_LEN, B, 1),
        hidden, embedding.astype(bf), enc_outs, ek,
        attn_W[:H].astype(bf), attn_v.reshape(1, H),
        dec_Wx[:E].astype(bf), dec_Wx[E:].astype(bf), dec_Wh.astype(bf),
        dec_bx.reshape(1, G3), dec_bh.reshape(1, G3),
        jnp.pad(fc_W.astype(bf), ((0, 0), (0, VPAD - V)))
           .reshape(E + 2 * H, NV, VT).transpose(1, 0, 2),
        jnp.pad(fc_b, (0, VPAD - V)).reshape(1, VPAD))
    return out


# speedup vs baseline: 1.1176x; 1.0024x over previous
import functools

import jax
import jax.numpy as jnp
from jax import lax
from jax.experimental import pallas as pl
from jax.experimental.pallas import tpu as pltpu
from jax.experimental.pallas import tpu_sc as plsc

V, E, H = 10000, 512, 512
SRC_LEN, TRG_LEN, B = 50, 50, 64
G3 = 3 * H

_NC, _NS = 2, 16
_NW = _NC * _NS
_NIDX = SRC_LEN * B
_NPAD = 3328
_BPW = _NPAD // _NW


@functools.cache
def _make_sc_gather():
    @functools.partial(
        pl.kernel,
        mesh=plsc.VectorSubcoreMesh(core_axis_name="c", subcore_axis_name="s",
                                    num_cores=_NC),
        out_type=jax.ShapeDtypeStruct((_NPAD, E), jnp.float32),
        scratch_types=[
            pltpu.VMEM((_BPW,), jnp.int32),
            pltpu.VMEM((_BPW, E), jnp.float32),
            pltpu.SemaphoreType.DMA,
        ],
    )
    def _sc_gather(table_hbm, idx_hbm, out_hbm, idx_v, rows_v, sem):
        wid = lax.axis_index("s") * _NC + lax.axis_index("c")
        base = wid * _BPW
        pltpu.sync_copy(idx_hbm.at[pl.ds(base, _BPW)], idx_v)
        pltpu.async_copy(table_hbm.at[idx_v], rows_v, sem).wait()
        pltpu.sync_copy(rows_v, out_hbm.at[pl.ds(base, _BPW)])

    return _sc_gather


def _dot(a, b):
    return lax.dot_general(a.astype(jnp.bfloat16), b.astype(jnp.bfloat16),
                           (((1,), (0,)), ((), ())),
                           preferred_element_type=jnp.float32,
                           precision=lax.Precision.DEFAULT)


def _enc_body(xemb_ref, wx_ref, wh_ref, bx_ref, bh_ref, wa2_ref,
              eo_ref, ek_ref, hn_ref, h_s, gx_s):
    h_s[...] = jnp.zeros((B, H), jnp.float32)
    gx_s[...] = _dot(xemb_ref[...].reshape(SRC_LEN * B, E), wx_ref[...]) \
        + bx_ref[...]
    wh = wh_ref[...]
    bh = bh_ref[...]

    def step(t, carry):
        h = h_s[...]
        gx = gx_s[pl.ds(t * B, B), :]
        gh = _dot(h, wh) + bh
        r = jax.nn.sigmoid(gx[:, :H] + gh[:, :H])
        z = jax.nn.sigmoid(gx[:, H:2 * H] + gh[:, H:2 * H])
        n = jnp.tanh(gx[:, 2 * H:] + r * gh[:, 2 * H:])
        h2 = (1.0 - z) * n + z * h
        eo_ref[t] = h2
        ek_ref[t] = _dot(h2, wa2_ref[...])
        h_s[...] = h2
        return carry

    lax.fori_loop(0, SRC_LEN, step, 0)
    hn_ref[...] = h_s[...]


def _encoder(src_emb, enc_Wx, enc_Wh, enc_bx, enc_bh, attn_W2):
    full = lambda s: pl.BlockSpec(s, lambda: tuple(0 for _ in s))
    return pl.pallas_call(
        _enc_body,
        grid=(),
        in_specs=[
            full((SRC_LEN, B, E)),
            full((E, G3)),
            full((H, G3)),
            full((1, G3)),
            full((1, G3)),
            full((H, H)),
        ],
        out_specs=[
            full((SRC_LEN, B, H)),
            full((SRC_LEN, B, H)),
            full((B, H)),
        ],
        out_shape=[
            jax.ShapeDtypeStruct((SRC_LEN, B, H), jnp.float32),
            jax.ShapeDtypeStruct((SRC_LEN, B, H), jnp.float32),
            jax.ShapeDtypeStruct((B, H), jnp.float32),
        ],
        scratch_shapes=[
            pltpu.VMEM((B, H), jnp.float32),
            pltpu.VMEM((SRC_LEN * B, G3), jnp.float32),
        ],
    )(src_emb, enc_Wx, enc_Wh, enc_bx, enc_bh, attn_W2)


VT = 2048
NV = (V + VT - 1) // VT
VPAD = NV * VT
_ECH = 2000
_NB = 4
_AHEAD = 3


def _dec_body(tf_ref, trg_ref, hid_ref, etab_ref, eo_ref, ek_ref,
              w1_ref, av_ref, wx_ref, wh_ref, bx_ref, bh_ref,
              fcb_ref, fcw_hbm, out_ref,
              h_s, feat_s, tok_s, bmax_s, barg_s, fbuf, sem):
    t = pl.program_id(0)

    def start(tile, slot):
        pltpu.make_async_copy(fcw_hbm.at[tile], fbuf.at[slot],
                              sem.at[slot]).start()

    def wait(slot):
        pltpu.make_async_copy(fcw_hbm.at[0], fbuf.at[slot],
                              sem.at[slot]).wait()

    @pl.when(t == 0)
    def _init():
        out_ref[...] = jnp.zeros_like(out_ref)
        h_s[...] = hid_ref[...]
        tok_s[...] = trg_ref[0]

    @pl.when(t > 0)
    def _step():
        for k in range(_AHEAD - 1):
            start(k, k)

        tok = tok_s[...]
        emb = jnp.zeros((B, E), jnp.float32)
        for c in range(V // _ECH):
            io = lax.broadcasted_iota(jnp.int32, (B, _ECH), 1) + c * _ECH
            oh = (io == tok).astype(jnp.float32)
            emb = emb + _dot(oh, etab_ref[c * _ECH:(c + 1) * _ECH, :])
        h = h_s[...]
        q = _dot(h, w1_ref[...])
        en = jnp.tanh(ek_ref[...] + q[None])
        sc = jnp.sum(en * av_ref[...][None], axis=2)
        m = jnp.max(sc, axis=0, keepdims=True)
        e = jnp.exp(sc - m)
        a = e / jnp.sum(e, axis=0, keepdims=True)
        ctx = jnp.sum(a[:, :, None] * eo_ref[...], axis=0)
        gx = _dot(jnp.concatenate([emb, ctx], axis=1), wx_ref[...]) \
            + bx_ref[...]
        gh = _dot(h, wh_ref[...]) + bh_ref[...]
        r = jax.nn.sigmoid(gx[:, :H] + gh[:, :H])
        z = jax.nn.sigmoid(gx[:, H:2 * H] + gh[:, H:2 * H])
        n = jnp.tanh(gx[:, 2 * H:] + r * gh[:, 2 * H:])
        h2 = (1.0 - z) * n + z * h
        h_s[...] = h2
        feat_s[:, :H] = h2
        feat_s[:, H:2 * H] = ctx
        feat_s[:, 2 * H:] = emb
        feat = feat_s[...]

        bmax = jnp.full((B, 1), -jnp.inf, jnp.float32)
        barg = jnp.zeros((B, 1), jnp.int32)
        for v in range(NV):
            if v + _AHEAD - 1 < NV:
                start(v + _AHEAD - 1, (v + _AHEAD - 1) % _NB)
            wait(v % _NB)
            w = VT if v < NV - 1 else V - (NV - 1) * VT
            logits = _dot(feat, fbuf[v % _NB]) \
                + fcb_ref[:, pl.ds(v * VT, VT)]
            logits = logits[:, :w]
            out_ref[0, :, pl.ds(v * VT, w)] = logits
            col = lax.broadcasted_iota(jnp.int32, (B, w), 1) + v * VT
            tmax = jnp.max(logits, axis=1, keepdims=True)
            targ = jnp.min(jnp.where(logits == tmax, col, jnp.int32(2 ** 30)),
                           axis=1, keepdims=True)
            better = tmax > bmax
            barg = jnp.where(better, targ, barg)
            bmax = jnp.maximum(tmax, bmax)
        bmax_s[...] = bmax
        barg_s[...] = barg

        tf = tf_ref[t]
        tok_s[...] = jnp.where(tf != 0, trg_ref[0], barg_s[...])


def _decoder(tf_i32, trg3, hidden, embedding, enc_outs, ek,
             attn_W1, attn_v2, dec_Wx2, dec_Wh, dec_bx2, dec_bh2,
             fcw3, fc_b2):
    cst = lambda s: pl.BlockSpec(s, lambda t, tf: tuple(0 for _ in s))
    grid_spec = pltpu.PrefetchScalarGridSpec(
        num_scalar_prefetch=1,
        grid=(TRG_LEN,),
        in_specs=[
            pl.BlockSpec((1, B, 1), lambda t, tf: (t, 0, 0)),
            cst((B, H)),
            cst((V, E)),
            cst((SRC_LEN, B, H)),
            cst((SRC_LEN, B, H)),
            cst((H, H)),
            cst((1, H)),
            cst((E + H, G3)),
            cst((H, G3)),
            cst((1, G3)),
            cst((1, G3)),
            cst((1, VPAD)),
            pl.BlockSpec(memory_space=pl.ANY),
        ],
        out_specs=pl.BlockSpec((1, B, V), lambda t, tf: (t, 0, 0)),
        scratch_shapes=[
            pltpu.VMEM((B, H), jnp.float32),
            pltpu.VMEM((B, E + 2 * H), jnp.float32),
            pltpu.VMEM((B, 1), jnp.int32),
            pltpu.VMEM((B, 1), jnp.float32),
            pltpu.VMEM((B, 1), jnp.int32),
            pltpu.VMEM((_NB, E + 2 * H, VT), fcw3.dtype),
            pltpu.SemaphoreType.DMA((_NB,)),
        ],
    )
    return pl.pallas_call(
        _dec_body,
        grid_spec=grid_spec,
        out_shape=jax.ShapeDtypeStruct((TRG_LEN, B, V), jnp.float32),
        compiler_params=pltpu.CompilerParams(
            dimension_semantics=("arbitrary",)),
    )(tf_i32, trg3, hidden, embedding, enc_outs, ek,
      attn_W1, attn_v2, dec_Wx2, dec_Wh, dec_bx2, dec_bh2, fc_b2, fcw3)


def kernel(src, trg, tf_flags, embedding, enc_Wx, enc_Wh, enc_bx, enc_bh,
           attn_W, attn_v, dec_Wx, dec_Wh, dec_bx, dec_bh, fc_W, fc_b):
    src_flat = src.reshape(-1).astype(jnp.int32)
    src_pad = jnp.pad(src_flat, (0, _NPAD - _NIDX))
    gathered = _make_sc_gather()(embedding, src_pad)
    src_emb = gathered[:_NIDX].reshape(SRC_LEN, B, E)

    bf = jnp.bfloat16
    enc_outs, ek, hidden = _encoder(
        src_emb, enc_Wx.astype(bf), enc_Wh.astype(bf),
        enc_bx.reshape(1, G3), enc_bh.reshape(1, G3), attn_W[H:].astype(bf))

    out = _decoder(
        tf_flags.astype(jnp.int32),
        trg.astype(jnp.int32).reshape(TRG_LEN, B, 1),
        hidden, embedding.astype(bf), enc_outs, ek,
        attn_W[:H].astype(bf), attn_v.reshape(1, H),
        dec_Wx.astype(bf), dec_Wh.astype(bf),
        dec_bx.reshape(1, G3), dec_bh.reshape(1, G3),
        jnp.pad(fc_W.astype(bf), ((0, 0), (0, VPAD - V)))
           .reshape(E + 2 * H, NV, VT).transpose(1, 0, 2),
        jnp.pad(fc_b, (0, VPAD - V)).reshape(1, VPAD))
    return out
